# Initial kernel scaffold; baseline (speedup 1.0000x reference)
import functools

import jax
import jax.numpy as jnp
from jax import lax
from jax.experimental import pallas as pl
from jax.experimental.pallas import tpu as pltpu
from jax.experimental.pallas import tpu_sc as plsc

_B, _C, _N, _K = 2, 9, 4096, 20
_BNSCALE = 1.0 / (1.0 + 1e-5) ** 0.5
_R = 256


def _lrelu(v):
    return jnp.where(v >= 0, v, 0.2 * v)


def _knn_body(xr_ref, xa_ref, idx_ref):
    xr = xr_ref[0]
    xa = xa_ref[0]
    g = lax.dot_general(xr, xa, (((1,), (1,)), ((), ())),
                        preferred_element_type=jnp.float32)
    sqr = jnp.sum(xr * xr, axis=1, keepdims=True)
    sqa = jnp.sum(xa * xa, axis=1)[None, :]
    d = (2.0 * g - sqr) - sqa
    it = lax.broadcasted_iota(jnp.int32, d.shape, 1)
    cols = []
    for _ in range(_K):
        m = jnp.max(d, axis=1, keepdims=True)
        am = jnp.min(jnp.where(d == m, it, _N), axis=1, keepdims=True)
        cols.append(am)
        d = jnp.where(it == am, -jnp.inf, d)
    idx_ref[0] = jnp.concatenate(cols, axis=1)


def _knn(xt):
    b, n, cp = xt.shape
    grid = (b, n // _R)
    return pl.pallas_call(
        _knn_body,
        grid=grid,
        in_specs=[
            pl.BlockSpec((1, _R, cp), lambda bb, i: (bb, i, 0)),
            pl.BlockSpec((1, n, cp), lambda bb, i: (bb, 0, 0)),
        ],
        out_specs=pl.BlockSpec((1, _R, _K), lambda bb, i: (bb, i, 0)),
        out_shape=jax.ShapeDtypeStruct((b, n, _K), jnp.int32),
    )(xt, xt)


def _pq_body(xt_ref, wa_ref, wq_ref, b_ref, p_ref, q_ref):
    xt = xt_ref[0]
    p_ref[0] = jnp.dot(xt, wa_ref[...], preferred_element_type=jnp.float32)
    q_ref[0] = (jnp.dot(xt, wq_ref[...], preferred_element_type=jnp.float32)
                + b_ref[...])


def _pq(xt, wa_t, wq_t, bias):
    b, n, cp = xt.shape
    o = wa_t.shape[1]
    grid = (b, n // _R)
    return pl.pallas_call(
        _pq_body,
        grid=grid,
        in_specs=[
            pl.BlockSpec((1, _R, cp), lambda bb, i: (bb, i, 0)),
            pl.BlockSpec((cp, o), lambda bb, i: (0, 0)),
            pl.BlockSpec((cp, o), lambda bb, i: (0, 0)),
            pl.BlockSpec((1, o), lambda bb, i: (0, 0)),
        ],
        out_specs=[
            pl.BlockSpec((1, _R, o), lambda bb, i: (bb, i, 0)),
            pl.BlockSpec((1, _R, o), lambda bb, i: (bb, i, 0)),
        ],
        out_shape=[
            jax.ShapeDtypeStruct((b, n, o), jnp.float32),
            jax.ShapeDtypeStruct((b, n, o), jnp.float32),
        ],
    )(xt, wa_t, wq_t, bias)


_NC, _NS = 2, 16
_NW = _NC * _NS
_CH = 128


def _sc_gather(table, idxg):
    m = idxg.shape[0]
    d = table.shape[1]
    per_w = m // _NW
    nch = per_w // _CH
    mesh = plsc.VectorSubcoreMesh(core_axis_name="c", subcore_axis_name="s")

    @functools.partial(
        pl.kernel,
        mesh=mesh,
        out_type=jax.ShapeDtypeStruct((m, d), jnp.float32),
        scratch_types=[
            pltpu.VMEM((_CH,), jnp.int32),
            pltpu.VMEM((_CH, d), jnp.float32),
            pltpu.SemaphoreType.DMA,
        ],
    )
    def k(table_hbm, idx_hbm, out_hbm, idx_v, rows_v, sem):
        wid = lax.axis_index("s") * _NC + lax.axis_index("c")
        base = wid * per_w

        def body(i, carry):
            off = base + i * _CH
            pltpu.sync_copy(idx_hbm.at[pl.ds(off, _CH)], idx_v)
            pltpu.async_copy(table_hbm.at[idx_v], rows_v, sem).wait()
            pltpu.sync_copy(rows_v, out_hbm.at[pl.ds(off, _CH)])
            return carry

        lax.fori_loop(0, nch, body, 0)

    return k(table, idxg)


def _edge_body(g_ref, q_ref, w_ref, b_ref, o_ref):
    g = g_ref[0]
    q = q_ref[0]
    h1 = _lrelu(g + q[None])
    h1f = h1.reshape(_K * _R, h1.shape[2])
    h2 = (jnp.dot(h1f, w_ref[...], preferred_element_type=jnp.float32)
          + b_ref[...])
    h2 = _lrelu(h2)
    o_ref[0] = jnp.max(h2.reshape(_K, _R, h2.shape[1]), axis=0)


def _edge(gath, q, w_t, bias):
    b, k, n, d = gath.shape
    o = w_t.shape[1]
    grid = (b, n // _R)
    return pl.pallas_call(
        _edge_body,
        grid=grid,
        in_specs=[
            pl.BlockSpec((1, k, _R, d), lambda bb, i: (bb, 0, i, 0)),
            pl.BlockSpec((1, _R, d), lambda bb, i: (bb, i, 0)),
            pl.BlockSpec((d, o), lambda bb, i: (0, 0)),
            pl.BlockSpec((1, o), lambda bb, i: (0, 0)),
        ],
        out_specs=pl.BlockSpec((1, _R, o), lambda bb, i: (bb, i, 0)),
        out_shape=jax.ShapeDtypeStruct((b, n, o), jnp.float32),
    )(gath, q, w_t, bias)


def _edge3_body(g_ref, q_ref, o_ref):
    g = g_ref[0]
    q = q_ref[0]
    o_ref[0] = jnp.max(_lrelu(g + q[None]), axis=0)


def _edge3(gath, q):
    b, k, n, d = gath.shape
    grid = (b, n // _R)
    return pl.pallas_call(
        _edge3_body,
        grid=grid,
        in_specs=[
            pl.BlockSpec((1, k, _R, d), lambda bb, i: (bb, 0, i, 0)),
            pl.BlockSpec((1, _R, d), lambda bb, i: (bb, i, 0)),
        ],
        out_specs=pl.BlockSpec((1, _R, d), lambda bb, i: (bb, i, 0)),
        out_shape=jax.ShapeDtypeStruct((b, n, d), jnp.float32),
    )(gath, q)


def _head6_body(xc_ref, w_ref, b_ref, o_ref):
    i = pl.program_id(1)
    y = (jnp.dot(xc_ref[0], w_ref[...], preferred_element_type=jnp.float32)
         + b_ref[...])
    y = _lrelu(y)
    pm = jnp.max(y, axis=0, keepdims=True)

    @pl.when(i == 0)
    def _():
        o_ref[0] = jnp.full_like(o_ref[0], -jnp.inf)

    o_ref[0] = jnp.maximum(o_ref[0], pm)


def _head6(xcat, w_t, bias):
    b, n, d = xcat.shape
    o = w_t.shape[1]
    grid = (b, n // _R)
    return pl.pallas_call(
        _head6_body,
        grid=grid,
        in_specs=[
            pl.BlockSpec((1, _R, d), lambda bb, i: (bb, i, 0)),
            pl.BlockSpec((d, o), lambda bb, i: (0, 0)),
            pl.BlockSpec((1, o), lambda bb, i: (0, 0)),
        ],
        out_specs=pl.BlockSpec((1, 1, o), lambda bb, i: (bb, 0, 0)),
        out_shape=jax.ShapeDtypeStruct((b, 1, o), jnp.float32),
    )(xcat, w_t, bias)


def _head789_body(hm_ref, xc_ref, w7h_ref, w7x_ref, b7_ref, w8_ref, b8_ref,
                  w9_ref, o_ref):
    hm = hm_ref[0]
    xc = xc_ref[0]
    y7 = (jnp.dot(xc, w7x_ref[...], preferred_element_type=jnp.float32)
          + jnp.dot(hm, w7h_ref[...], preferred_element_type=jnp.float32)
          + b7_ref[...])
    y7 = _lrelu(y7)
    y8 = _lrelu(jnp.dot(y7, w8_ref[...], preferred_element_type=jnp.float32)
                + b8_ref[...])
    o_ref[0] = jnp.dot(y8, w9_ref[...], preferred_element_type=jnp.float32)


def _head789(hm, xcat, w7h, w7x, b7, w8, b8, w9p):
    b, n, d = xcat.shape
    emb = hm.shape[2]
    o9 = w9p.shape[1]
    grid = (b, n // _R)
    return pl.pallas_call(
        _head789_body,
        grid=grid,
        in_specs=[
            pl.BlockSpec((1, 1, emb), lambda bb, i: (bb, 0, 0)),
            pl.BlockSpec((1, _R, d), lambda bb, i: (bb, i, 0)),
            pl.BlockSpec((emb, 512), lambda bb, i: (0, 0)),
            pl.BlockSpec((d, 512), lambda bb, i: (0, 0)),
            pl.BlockSpec((1, 512), lambda bb, i: (0, 0)),
            pl.BlockSpec((512, 256), lambda bb, i: (0, 0)),
            pl.BlockSpec((1, 256), lambda bb, i: (0, 0)),
            pl.BlockSpec((256, o9), lambda bb, i: (0, 0)),
        ],
        out_specs=pl.BlockSpec((1, _R, o9), lambda bb, i: (bb, i, 0)),
        out_shape=jax.ShapeDtypeStruct((b, n, o9), jnp.float32),
    )(hm, xcat, w7h, w7x, b7, w8, b8, w9p)


def _fold(w, g):
    return w * (g * _BNSCALE)[:, None]


def _gather_indices(idx):
    b = idx.shape[0]
    idx_t = jnp.transpose(idx, (0, 2, 1))
    idx_t = idx_t + (jnp.arange(b, dtype=jnp.int32) * _N)[:, None, None]
    return idx_t.reshape(-1)


def _edge_layer(feats, wa_t, wq_t, bias):
    b, n, d = feats.shape
    idx = _knn(feats)
    p, q = _pq(feats, wa_t, wq_t, bias)
    gath = _sc_gather(p.reshape(b * n, p.shape[2]), _gather_indices(idx))
    return gath.reshape(b, _K, n, p.shape[2]), q


def kernel(x, W1, g1, b1, W2, g2, b2, W3, g3, b3, W4, g4, b4, W5, g5, b5,
           W6, g6, b6, W7, g7, b7, W8, g8, b8, W9):
    f = jnp.float32
    W1f, W2f = _fold(W1, g1), _fold(W2, g2)
    W3f, W4f = _fold(W3, g3), _fold(W4, g4)
    W5f, W6f = _fold(W5, g5), _fold(W6, g6)
    W7f, W8f = _fold(W7, g7), _fold(W8, g8)

    xt = jnp.transpose(x, (0, 2, 1)).astype(f)
    cp = 16
    xt_p = jnp.pad(xt, ((0, 0), (0, 0), (0, cp - _C)))
    wa1 = jnp.pad(W1f[:, :_C].T, ((0, cp - _C), (0, 0)))
    wq1 = jnp.pad((W1f[:, _C:] - W1f[:, :_C]).T, ((0, cp - _C), (0, 0)))
    gath1, q1 = _edge_layer(xt_p, wa1, wq1, b1[None, :])
    x1 = _edge(gath1, q1, W2f.T, b2[None, :])

    wa3 = W3f[:, :64].T
    wq3 = (W3f[:, 64:] - W3f[:, :64]).T
    gath2, q3 = _edge_layer(x1, wa3, wq3, b3[None, :])
    x2 = _edge(gath2, q3, W4f.T, b4[None, :])

    wa5 = W5f[:, :64].T
    wq5 = (W5f[:, 64:] - W5f[:, :64]).T
    gath3, q5 = _edge_layer(x2, wa5, wq5, b5[None, :])
    x3 = _edge3(gath3, q5)

    xcat = jnp.concatenate([x1, x2, x3], axis=-1)
    hm = _head6(xcat, W6f.T, b6[None, :])
    w7h = W7f[:, :1024].T
    w7x = W7f[:, 1024:].T
    o9 = 128
    w9p = jnp.pad(W9.T, ((0, 0), (0, o9 - W9.shape[0])))
    out = _head789(hm, xcat, w7h, w7x, b7[None, :], W8f.T, b8[None, :], w9p)
    return jnp.transpose(out[:, :, :W9.shape[0]], (0, 2, 1))


# speedup vs baseline: 11.3190x; 11.3190x over previous
import functools

import jax
import jax.numpy as jnp
from jax import lax
from jax.experimental import pallas as pl
from jax.experimental.pallas import tpu as pltpu
from jax.experimental.pallas import tpu_sc as plsc

_B, _C, _N, _K = 2, 9, 4096, 20
_BNSCALE = 1.0 / (1.0 + 1e-5) ** 0.5
_R = 256


def _lrelu(v):
    return jnp.where(v >= 0, v, 0.2 * v)


def _knn_body(xr_ref, xa_ref, idx_ref):
    xr = xr_ref[0]
    xa = xa_ref[0]
    g = lax.dot_general(xr, xa, (((1,), (1,)), ((), ())),
                        preferred_element_type=jnp.float32)
    sqr = jnp.sum(xr * xr, axis=1, keepdims=True)
    sqa = jnp.sum(xa * xa, axis=1)[None, :]
    d = (2.0 * g - sqr) - sqa
    it = lax.broadcasted_iota(jnp.int32, d.shape, 1)
    cols = []
    for _ in range(_K):
        m = jnp.max(d, axis=1, keepdims=True)
        am = jnp.min(jnp.where(d == m, it, _N), axis=1, keepdims=True)
        cols.append(am)
        d = jnp.where(it == am, -jnp.inf, d)
    idx_ref[0] = jnp.concatenate(cols, axis=1)


def _knn(xt):
    b, n, cp = xt.shape
    grid = (b, n // _R)
    return pl.pallas_call(
        _knn_body,
        grid=grid,
        in_specs=[
            pl.BlockSpec((1, _R, cp), lambda bb, i: (bb, i, 0)),
            pl.BlockSpec((1, n, cp), lambda bb, i: (bb, 0, 0)),
        ],
        out_specs=pl.BlockSpec((1, _R, _K), lambda bb, i: (bb, i, 0)),
        out_shape=jax.ShapeDtypeStruct((b, n, _K), jnp.int32),
    )(xt, xt)


def _pq_body(xt_ref, wa_ref, wq_ref, b_ref, p_ref, q_ref):
    xt = xt_ref[0]
    p_ref[0] = jnp.dot(xt, wa_ref[...], preferred_element_type=jnp.float32)
    q_ref[0] = (jnp.dot(xt, wq_ref[...], preferred_element_type=jnp.float32)
                + b_ref[...])


def _pq(xt, wa_t, wq_t, bias):
    b, n, cp = xt.shape
    op = wa_t.shape[1]
    o = wq_t.shape[1]
    grid = (b, n // _R)
    return pl.pallas_call(
        _pq_body,
        grid=grid,
        in_specs=[
            pl.BlockSpec((1, _R, cp), lambda bb, i: (bb, i, 0)),
            pl.BlockSpec((cp, op), lambda bb, i: (0, 0)),
            pl.BlockSpec((cp, o), lambda bb, i: (0, 0)),
            pl.BlockSpec((1, o), lambda bb, i: (0, 0)),
        ],
        out_specs=[
            pl.BlockSpec((1, _R, op), lambda bb, i: (bb, i, 0)),
            pl.BlockSpec((1, _R, o), lambda bb, i: (bb, i, 0)),
        ],
        out_shape=[
            jax.ShapeDtypeStruct((b, n, op), jnp.float32),
            jax.ShapeDtypeStruct((b, n, o), jnp.float32),
        ],
    )(xt, wa_t, wq_t, bias)


_NC, _NS = 2, 16
_NW = _NC * _NS
_CH = 128


def _sc_gather(table, idxg):
    m = idxg.shape[0]
    d = table.shape[1]
    per_w = m // _NW
    nch = per_w // _CH
    mesh = plsc.VectorSubcoreMesh(core_axis_name="c", subcore_axis_name="s")

    @functools.partial(
        pl.kernel,
        mesh=mesh,
        out_type=jax.ShapeDtypeStruct((m, d), jnp.float32),
        scratch_types=[
            pltpu.VMEM((_CH,), jnp.int32),
            pltpu.VMEM((_CH, d), jnp.float32),
            pltpu.SemaphoreType.DMA,
        ],
    )
    def k(table_hbm, idx_hbm, out_hbm, idx_v, rows_v, sem):
        wid = lax.axis_index("s") * _NC + lax.axis_index("c")
        base = wid * per_w

        def body(i, carry):
            off = base + i * _CH
            pltpu.sync_copy(idx_hbm.at[pl.ds(off, _CH)], idx_v)
            pltpu.async_copy(table_hbm.at[idx_v], rows_v, sem).wait()
            pltpu.sync_copy(rows_v, out_hbm.at[pl.ds(off, _CH)])
            return carry

        lax.fori_loop(0, nch, body, 0)

    return k(table, idxg)


def _edge_body(g_ref, q_ref, w_ref, b_ref, o_ref):
    q = q_ref[0]
    g = g_ref[0][:, :, :q.shape[1]]
    h1 = _lrelu(g + q[None])
    h1f = h1.reshape(_K * _R, h1.shape[2])
    h2 = (jnp.dot(h1f, w_ref[...], preferred_element_type=jnp.float32)
          + b_ref[...])
    h2 = _lrelu(h2)
    o_ref[0] = jnp.max(h2.reshape(_K, _R, h2.shape[1]), axis=0)


def _edge(gath, q, w_t, bias):
    b, k, n, gp = gath.shape
    d = q.shape[2]
    o = w_t.shape[1]
    grid = (b, n // _R)
    return pl.pallas_call(
        _edge_body,
        grid=grid,
        in_specs=[
            pl.BlockSpec((1, k, _R, gp), lambda bb, i: (bb, 0, i, 0)),
            pl.BlockSpec((1, _R, d), lambda bb, i: (bb, i, 0)),
            pl.BlockSpec((d, o), lambda bb, i: (0, 0)),
            pl.BlockSpec((1, o), lambda bb, i: (0, 0)),
        ],
        out_specs=pl.BlockSpec((1, _R, o), lambda bb, i: (bb, i, 0)),
        out_shape=jax.ShapeDtypeStruct((b, n, o), jnp.float32),
    )(gath, q, w_t, bias)


def _edge3_body(g_ref, q_ref, o_ref):
    q = q_ref[0]
    g = g_ref[0][:, :, :q.shape[1]]
    o_ref[0] = jnp.max(_lrelu(g + q[None]), axis=0)


def _edge3(gath, q):
    b, k, n, gp = gath.shape
    d = q.shape[2]
    grid = (b, n // _R)
    return pl.pallas_call(
        _edge3_body,
        grid=grid,
        in_specs=[
            pl.BlockSpec((1, k, _R, gp), lambda bb, i: (bb, 0, i, 0)),
            pl.BlockSpec((1, _R, d), lambda bb, i: (bb, i, 0)),
        ],
        out_specs=pl.BlockSpec((1, _R, d), lambda bb, i: (bb, i, 0)),
        out_shape=jax.ShapeDtypeStruct((b, n, d), jnp.float32),
    )(gath, q)


def _head6_body(xc_ref, w_ref, b_ref, o_ref):
    i = pl.program_id(1)
    y = (jnp.dot(xc_ref[0], w_ref[...], preferred_element_type=jnp.float32)
         + b_ref[...])
    y = _lrelu(y)
    pm = jnp.max(y, axis=0, keepdims=True)

    @pl.when(i == 0)
    def _():
        o_ref[0] = jnp.full_like(o_ref[0], -jnp.inf)

    o_ref[0] = jnp.maximum(o_ref[0], pm)


def _head6(xcat, w_t, bias):
    b, n, d = xcat.shape
    o = w_t.shape[1]
    grid = (b, n // _R)
    return pl.pallas_call(
        _head6_body,
        grid=grid,
        in_specs=[
            pl.BlockSpec((1, _R, d), lambda bb, i: (bb, i, 0)),
            pl.BlockSpec((d, o), lambda bb, i: (0, 0)),
            pl.BlockSpec((1, o), lambda bb, i: (0, 0)),
        ],
        out_specs=pl.BlockSpec((1, 1, o), lambda bb, i: (bb, 0, 0)),
        out_shape=jax.ShapeDtypeStruct((b, 1, o), jnp.float32),
    )(xcat, w_t, bias)


def _head789_body(hm_ref, xc_ref, w7h_ref, w7x_ref, b7_ref, w8_ref, b8_ref,
                  w9_ref, o_ref):
    hm = hm_ref[0]
    xc = xc_ref[0]
    y7 = (jnp.dot(xc, w7x_ref[...], preferred_element_type=jnp.float32)
          + jnp.dot(hm, w7h_ref[...], preferred_element_type=jnp.float32)
          + b7_ref[...])
    y7 = _lrelu(y7)
    y8 = _lrelu(jnp.dot(y7, w8_ref[...], preferred_element_type=jnp.float32)
                + b8_ref[...])
    o_ref[0] = jnp.dot(y8, w9_ref[...], preferred_element_type=jnp.float32)


def _head789(hm, xcat, w7h, w7x, b7, w8, b8, w9p):
    b, n, d = xcat.shape
    emb = hm.shape[2]
    o9 = w9p.shape[1]
    grid = (b, n // _R)
    return pl.pallas_call(
        _head789_body,
        grid=grid,
        in_specs=[
            pl.BlockSpec((1, 1, emb), lambda bb, i: (bb, 0, 0)),
            pl.BlockSpec((1, _R, d), lambda bb, i: (bb, i, 0)),
            pl.BlockSpec((emb, 512), lambda bb, i: (0, 0)),
            pl.BlockSpec((d, 512), lambda bb, i: (0, 0)),
            pl.BlockSpec((1, 512), lambda bb, i: (0, 0)),
            pl.BlockSpec((512, 256), lambda bb, i: (0, 0)),
            pl.BlockSpec((1, 256), lambda bb, i: (0, 0)),
            pl.BlockSpec((256, o9), lambda bb, i: (0, 0)),
        ],
        out_specs=pl.BlockSpec((1, _R, o9), lambda bb, i: (bb, i, 0)),
        out_shape=jax.ShapeDtypeStruct((b, n, o9), jnp.float32),
    )(hm, xcat, w7h, w7x, b7, w8, b8, w9p)


def _fold(w, g):
    return w * (g * _BNSCALE)[:, None]


def _gather_indices(idx):
    b = idx.shape[0]
    idx_t = jnp.transpose(idx, (0, 2, 1))
    idx_t = idx_t + (jnp.arange(b, dtype=jnp.int32) * _N)[:, None, None]
    return idx_t.reshape(-1)


def _edge_layer(feats, wa_t, wq_t, bias):
    b, n, d = feats.shape
    idx = _knn(feats)
    wa_p = jnp.pad(wa_t, ((0, 0), (0, 128 - wa_t.shape[1])))
    p, q = _pq(feats, wa_p, wq_t, bias)
    gath = _sc_gather(p.reshape(b * n, p.shape[2]), _gather_indices(idx))
    return gath.reshape(b, _K, n, p.shape[2]), q


def kernel(x, W1, g1, b1, W2, g2, b2, W3, g3, b3, W4, g4, b4, W5, g5, b5,
           W6, g6, b6, W7, g7, b7, W8, g8, b8, W9):
    f = jnp.float32
    W1f, W2f = _fold(W1, g1), _fold(W2, g2)
    W3f, W4f = _fold(W3, g3), _fold(W4, g4)
    W5f, W6f = _fold(W5, g5), _fold(W6, g6)
    W7f, W8f = _fold(W7, g7), _fold(W8, g8)

    xt = jnp.transpose(x, (0, 2, 1)).astype(f)
    cp = 16
    xt_p = jnp.pad(xt, ((0, 0), (0, 0), (0, cp - _C)))
    wa1 = jnp.pad(W1f[:, :_C].T, ((0, cp - _C), (0, 0)))
    wq1 = jnp.pad((W1f[:, _C:] - W1f[:, :_C]).T, ((0, cp - _C), (0, 0)))
    gath1, q1 = _edge_layer(xt_p, wa1, wq1, b1[None, :])
    x1 = _edge(gath1, q1, W2f.T, b2[None, :])

    wa3 = W3f[:, :64].T
    wq3 = (W3f[:, 64:] - W3f[:, :64]).T
    gath2, q3 = _edge_layer(x1, wa3, wq3, b3[None, :])
    x2 = _edge(gath2, q3, W4f.T, b4[None, :])

    wa5 = W5f[:, :64].T
    wq5 = (W5f[:, 64:] - W5f[:, :64]).T
    gath3, q5 = _edge_layer(x2, wa5, wq5, b5[None, :])
    x3 = _edge3(gath3, q5)

    xcat = jnp.concatenate([x1, x2, x3], axis=-1)
    hm = _head6(xcat, W6f.T, b6[None, :])
    w7h = W7f[:, :1024].T
    w7x = W7f[:, 1024:].T
    o9 = 128
    w9p = jnp.pad(W9.T, ((0, 0), (0, o9 - W9.shape[0])))
    out = _head789(hm, xcat, w7h, w7x, b7[None, :], W8f.T, b8[None, :], w9p)
    return jnp.transpose(out[:, :, :W9.shape[0]], (0, 2, 1))


# speedup vs baseline: 13.4568x; 1.1889x over previous
import functools

import jax
import jax.numpy as jnp
from jax import lax
from jax.experimental import pallas as pl
from jax.experimental.pallas import tpu as pltpu
from jax.experimental.pallas import tpu_sc as plsc

_B, _C, _N, _K = 2, 9, 4096, 20
_BNSCALE = 1.0 / (1.0 + 1e-5) ** 0.5
_R = 256


def _lrelu(v):
    return jnp.where(v >= 0, v, 0.2 * v)


def _knn_body(xr_ref, xa_ref, idx_ref):
    xr = xr_ref[0]
    xa = xa_ref[0]
    g = lax.dot_general(xr, xa, (((1,), (1,)), ((), ())),
                        preferred_element_type=jnp.float32)
    sqr = jnp.sum(xr * xr, axis=1, keepdims=True)
    sqa = jnp.sum(xa * xa, axis=1)[None, :]
    d = (2.0 * g - sqr) - sqa
    it = lax.broadcasted_iota(jnp.int32, d.shape, 1)
    cols = []
    for _ in range(_K):
        am = jnp.argmax(d, axis=1)[:, None]
        cols.append(am)
        d = jnp.where(it == am, -jnp.inf, d)
    idx_ref[0] = jnp.concatenate(cols, axis=1)


def _knn(xt):
    b, n, cp = xt.shape
    grid = (b, n // _R)
    return pl.pallas_call(
        _knn_body,
        grid=grid,
        in_specs=[
            pl.BlockSpec((1, _R, cp), lambda bb, i: (bb, i, 0)),
            pl.BlockSpec((1, n, cp), lambda bb, i: (bb, 0, 0)),
        ],
        out_specs=pl.BlockSpec((1, _R, _K), lambda bb, i: (bb, i, 0)),
        out_shape=jax.ShapeDtypeStruct((b, n, _K), jnp.int32),
    )(xt, xt)


def _pq_body(xt_ref, wa_ref, wq_ref, b_ref, p_ref, q_ref):
    xt = xt_ref[0]
    p_ref[0] = jnp.dot(xt, wa_ref[...], preferred_element_type=jnp.float32)
    q_ref[0] = (jnp.dot(xt, wq_ref[...], preferred_element_type=jnp.float32)
                + b_ref[...])


def _pq(xt, wa_t, wq_t, bias):
    b, n, cp = xt.shape
    op = wa_t.shape[1]
    o = wq_t.shape[1]
    grid = (b, n // _R)
    return pl.pallas_call(
        _pq_body,
        grid=grid,
        in_specs=[
            pl.BlockSpec((1, _R, cp), lambda bb, i: (bb, i, 0)),
            pl.BlockSpec((cp, op), lambda bb, i: (0, 0)),
            pl.BlockSpec((cp, o), lambda bb, i: (0, 0)),
            pl.BlockSpec((1, o), lambda bb, i: (0, 0)),
        ],
        out_specs=[
            pl.BlockSpec((1, _R, op), lambda bb, i: (bb, i, 0)),
            pl.BlockSpec((1, _R, o), lambda bb, i: (bb, i, 0)),
        ],
        out_shape=[
            jax.ShapeDtypeStruct((b, n, op), jnp.float32),
            jax.ShapeDtypeStruct((b, n, o), jnp.float32),
        ],
    )(xt, wa_t, wq_t, bias)


_NC, _NS = 2, 16
_NW = _NC * _NS
_CH = 128


def _sc_gather(table, idxg):
    m = idxg.shape[0]
    d = table.shape[1]
    per_w = m // _NW
    nch = per_w // _CH
    mesh = plsc.VectorSubcoreMesh(core_axis_name="c", subcore_axis_name="s")

    @functools.partial(
        pl.kernel,
        mesh=mesh,
        out_type=jax.ShapeDtypeStruct((m, d), jnp.float32),
        scratch_types=[
            pltpu.VMEM((2, _CH), jnp.int32),
            pltpu.VMEM((2, _CH, d), jnp.float32),
            pltpu.SemaphoreType.DMA,
            pltpu.SemaphoreType.DMA,
        ],
    )
    def k(table_hbm, idx_hbm, out_hbm, idx_v, rows_v, sem0, sem1):
        wid = lax.axis_index("s") * _NC + lax.axis_index("c")
        base = wid * per_w
        sems = (sem0, sem1)

        pltpu.sync_copy(idx_hbm.at[pl.ds(base, _CH)], idx_v.at[0])
        pltpu.async_copy(table_hbm.at[idx_v.at[0]], rows_v.at[0], sem0)

        assert nch % 2 == 0

        def body(g2, carry):
            for bslot in (0, 1):
                g = g2 * 2 + bslot
                nxt = 1 - bslot

                @pl.when(g + 1 < nch)
                def _():
                    off_n = base + (g + 1) * _CH
                    pltpu.sync_copy(idx_hbm.at[pl.ds(off_n, _CH)],
                                    idx_v.at[nxt])
                    pltpu.async_copy(table_hbm.at[idx_v.at[nxt]],
                                     rows_v.at[nxt], sems[nxt])

                pltpu.make_async_copy(table_hbm.at[idx_v.at[bslot]],
                                      rows_v.at[bslot], sems[bslot]).wait()
                off = base + g * _CH
                pltpu.sync_copy(rows_v.at[bslot], out_hbm.at[pl.ds(off, _CH)])
            return carry

        lax.fori_loop(0, nch // 2, body, 0)

    return k(table, idxg)


def _edge_body(g_ref, q_ref, w_ref, b_ref, o_ref):
    q = q_ref[0]
    g = g_ref[0][:, :, :q.shape[1]]
    h1 = _lrelu(g + q[None])
    h1f = h1.reshape(_K * _R, h1.shape[2])
    h2 = (jnp.dot(h1f, w_ref[...], preferred_element_type=jnp.float32)
          + b_ref[...])
    h2 = _lrelu(h2)
    o_ref[0] = jnp.max(h2.reshape(_K, _R, h2.shape[1]), axis=0)


def _edge(gath, q, w_t, bias):
    b, k, n, gp = gath.shape
    d = q.shape[2]
    o = w_t.shape[1]
    grid = (b, n // _R)
    return pl.pallas_call(
        _edge_body,
        grid=grid,
        in_specs=[
            pl.BlockSpec((1, k, _R, gp), lambda bb, i: (bb, 0, i, 0)),
            pl.BlockSpec((1, _R, d), lambda bb, i: (bb, i, 0)),
            pl.BlockSpec((d, o), lambda bb, i: (0, 0)),
            pl.BlockSpec((1, o), lambda bb, i: (0, 0)),
        ],
        out_specs=pl.BlockSpec((1, _R, o), lambda bb, i: (bb, i, 0)),
        out_shape=jax.ShapeDtypeStruct((b, n, o), jnp.float32),
    )(gath, q, w_t, bias)


def _edge3_body(g_ref, q_ref, o_ref):
    q = q_ref[0]
    g = g_ref[0][:, :, :q.shape[1]]
    o_ref[0] = jnp.max(_lrelu(g + q[None]), axis=0)


def _edge3(gath, q):
    b, k, n, gp = gath.shape
    d = q.shape[2]
    grid = (b, n // _R)
    return pl.pallas_call(
        _edge3_body,
        grid=grid,
        in_specs=[
            pl.BlockSpec((1, k, _R, gp), lambda bb, i: (bb, 0, i, 0)),
            pl.BlockSpec((1, _R, d), lambda bb, i: (bb, i, 0)),
        ],
        out_specs=pl.BlockSpec((1, _R, d), lambda bb, i: (bb, i, 0)),
        out_shape=jax.ShapeDtypeStruct((b, n, d), jnp.float32),
    )(gath, q)


def _head6_body(xc_ref, w_ref, b_ref, o_ref):
    i = pl.program_id(1)
    y = (jnp.dot(xc_ref[0], w_ref[...], preferred_element_type=jnp.float32)
         + b_ref[...])
    y = _lrelu(y)
    pm = jnp.max(y, axis=0, keepdims=True)

    @pl.when(i == 0)
    def _():
        o_ref[0] = jnp.full_like(o_ref[0], -jnp.inf)

    o_ref[0] = jnp.maximum(o_ref[0], pm)


def _head6(xcat, w_t, bias):
    b, n, d = xcat.shape
    o = w_t.shape[1]
    grid = (b, n // _R)
    return pl.pallas_call(
        _head6_body,
        grid=grid,
        in_specs=[
            pl.BlockSpec((1, _R, d), lambda bb, i: (bb, i, 0)),
            pl.BlockSpec((d, o), lambda bb, i: (0, 0)),
            pl.BlockSpec((1, o), lambda bb, i: (0, 0)),
        ],
        out_specs=pl.BlockSpec((1, 1, o), lambda bb, i: (bb, 0, 0)),
        out_shape=jax.ShapeDtypeStruct((b, 1, o), jnp.float32),
    )(xcat, w_t, bias)


def _head789_body(hm_ref, xc_ref, w7h_ref, w7x_ref, b7_ref, w8_ref, b8_ref,
                  w9_ref, o_ref):
    hm = hm_ref[0]
    xc = xc_ref[0]
    y7 = (jnp.dot(xc, w7x_ref[...], preferred_element_type=jnp.float32)
          + jnp.dot(hm, w7h_ref[...], preferred_element_type=jnp.float32)
          + b7_ref[...])
    y7 = _lrelu(y7)
    y8 = _lrelu(jnp.dot(y7, w8_ref[...], preferred_element_type=jnp.float32)
                + b8_ref[...])
    o_ref[0] = jnp.dot(y8, w9_ref[...], preferred_element_type=jnp.float32)


def _head789(hm, xcat, w7h, w7x, b7, w8, b8, w9p):
    b, n, d = xcat.shape
    emb = hm.shape[2]
    o9 = w9p.shape[1]
    grid = (b, n // _R)
    return pl.pallas_call(
        _head789_body,
        grid=grid,
        in_specs=[
            pl.BlockSpec((1, 1, emb), lambda bb, i: (bb, 0, 0)),
            pl.BlockSpec((1, _R, d), lambda bb, i: (bb, i, 0)),
            pl.BlockSpec((emb, 512), lambda bb, i: (0, 0)),
            pl.BlockSpec((d, 512), lambda bb, i: (0, 0)),
            pl.BlockSpec((1, 512), lambda bb, i: (0, 0)),
            pl.BlockSpec((512, 256), lambda bb, i: (0, 0)),
            pl.BlockSpec((1, 256), lambda bb, i: (0, 0)),
            pl.BlockSpec((256, o9), lambda bb, i: (0, 0)),
        ],
        out_specs=pl.BlockSpec((1, _R, o9), lambda bb, i: (bb, i, 0)),
        out_shape=jax.ShapeDtypeStruct((b, n, o9), jnp.float32),
    )(hm, xcat, w7h, w7x, b7, w8, b8, w9p)


def _fold(w, g):
    return w * (g * _BNSCALE)[:, None]


def _gather_indices(idx):
    b = idx.shape[0]
    idx_t = jnp.transpose(idx, (0, 2, 1))
    idx_t = idx_t + (jnp.arange(b, dtype=jnp.int32) * _N)[:, None, None]
    return idx_t.reshape(-1)


def _edge_layer(feats, wa_t, wq_t, bias):
    b, n, d = feats.shape
    idx = _knn(feats)
    wa_p = jnp.pad(wa_t, ((0, 0), (0, 128 - wa_t.shape[1])))
    p, q = _pq(feats, wa_p, wq_t, bias)
    gath = _sc_gather(p.reshape(b * n, p.shape[2]), _gather_indices(idx))
    return gath.reshape(b, _K, n, p.shape[2]), q


def kernel(x, W1, g1, b1, W2, g2, b2, W3, g3, b3, W4, g4, b4, W5, g5, b5,
           W6, g6, b6, W7, g7, b7, W8, g8, b8, W9):
    f = jnp.float32
    W1f, W2f = _fold(W1, g1), _fold(W2, g2)
    W3f, W4f = _fold(W3, g3), _fold(W4, g4)
    W5f, W6f = _fold(W5, g5), _fold(W6, g6)
    W7f, W8f = _fold(W7, g7), _fold(W8, g8)

    xt = jnp.transpose(x, (0, 2, 1)).astype(f)
    cp = 16
    xt_p = jnp.pad(xt, ((0, 0), (0, 0), (0, cp - _C)))
    wa1 = jnp.pad(W1f[:, :_C].T, ((0, cp - _C), (0, 0)))
    wq1 = jnp.pad((W1f[:, _C:] - W1f[:, :_C]).T, ((0, cp - _C), (0, 0)))
    gath1, q1 = _edge_layer(xt_p, wa1, wq1, b1[None, :])
    x1 = _edge(gath1, q1, W2f.T, b2[None, :])

    wa3 = W3f[:, :64].T
    wq3 = (W3f[:, 64:] - W3f[:, :64]).T
    gath2, q3 = _edge_layer(x1, wa3, wq3, b3[None, :])
    x2 = _edge(gath2, q3, W4f.T, b4[None, :])

    wa5 = W5f[:, :64].T
    wq5 = (W5f[:, 64:] - W5f[:, :64]).T
    gath3, q5 = _edge_layer(x2, wa5, wq5, b5[None, :])
    x3 = _edge3(gath3, q5)

    xcat = jnp.concatenate([x1, x2, x3], axis=-1)
    hm = _head6(xcat, W6f.T, b6[None, :])
    w7h = W7f[:, :1024].T
    w7x = W7f[:, 1024:].T
    o9 = 128
    w9p = jnp.pad(W9.T, ((0, 0), (0, o9 - W9.shape[0])))
    out = _head789(hm, xcat, w7h, w7x, b7[None, :], W8f.T, b8[None, :], w9p)
    return jnp.transpose(out[:, :, :W9.shape[0]], (0, 2, 1))
